# baseline (speedup 1.0000x reference)
import jax
import jax.numpy as jnp
from jax.experimental import pallas as pl

_HIDDEN = 256
_NUM_EXPERTS = 8
_TB = 1024


def _moe_block_kernel(x_ref, wg_ref, we_ref, be_ref, out_ref):
    xb = x_ref[...]
    logits = jnp.dot(xb, wg_ref[...], preferred_element_type=jnp.float32)
    m = jnp.max(logits, axis=-1, keepdims=True)
    e = jnp.exp(logits - m)
    s = jnp.sum(e, axis=-1, keepdims=True)
    gates = e / s
    idx = jnp.argmax(logits, axis=-1)
    gate_val = jnp.max(gates, axis=-1)

    acc = jnp.zeros((_TB, _HIDDEN), dtype=jnp.float32)
    xb16 = xb.astype(jnp.bfloat16)
    for ex in range(_NUM_EXPERTS):
        y = jnp.dot(xb16, we_ref[ex].astype(jnp.bfloat16),
                    preferred_element_type=jnp.float32)
        mask = (idx == ex)[:, None]
        acc = acc + jnp.where(mask, y, 0.0)
    oh = (idx[:, None] ==
          jax.lax.broadcasted_iota(jnp.int32, (_TB, _NUM_EXPERTS), 1))
    bias = jnp.dot(oh.astype(jnp.float32), be_ref[...],
                   preferred_element_type=jnp.float32)
    out_ref[...] = gate_val[:, None] * (acc + bias)


@jax.jit
def kernel(x, Wg, We, be):
    B, S, H = x.shape
    T = B * S
    xt = x.reshape(T, H)
    grid = (T // _TB,)
    out = pl.pallas_call(
        _moe_block_kernel,
        grid=grid,
        in_specs=[
            pl.BlockSpec((_TB, H), lambda i: (i, 0)),
            pl.BlockSpec((H, _NUM_EXPERTS), lambda i: (0, 0)),
            pl.BlockSpec((_NUM_EXPERTS, H, H), lambda i: (0, 0, 0)),
            pl.BlockSpec((_NUM_EXPERTS, H), lambda i: (0, 0)),
        ],
        out_specs=pl.BlockSpec((_TB, H), lambda i: (i, 0)),
        out_shape=jax.ShapeDtypeStruct((T, H), jnp.float32),
    )(xt, Wg, We, be)
    return out.reshape(B, S, H)


# speedup vs baseline: 1.0567x; 1.0567x over previous
import jax
import jax.numpy as jnp
from jax.experimental import pallas as pl

_HIDDEN = 256
_NUM_EXPERTS = 8
_TB = 2048


def _moe_block_kernel(x_ref, wg_ref, we_ref, be_ref, out_ref):
    xb = x_ref[...]
    logits = jnp.dot(xb, wg_ref[...], preferred_element_type=jnp.float32)
    m = jnp.max(logits, axis=-1, keepdims=True)
    e = jnp.exp(logits - m)
    s = jnp.sum(e, axis=-1, keepdims=True)
    gates = e / s
    idx = jnp.argmax(logits, axis=-1)
    gate_val = jnp.max(gates, axis=-1)

    xb16 = xb.astype(jnp.bfloat16)
    acc = jnp.dot(xb16, we_ref[0].astype(jnp.bfloat16),
                  preferred_element_type=jnp.float32)
    for ex in range(1, _NUM_EXPERTS):
        y = jnp.dot(xb16, we_ref[ex].astype(jnp.bfloat16),
                    preferred_element_type=jnp.float32)
        mask = (idx == ex)[:, None]
        acc = jnp.where(mask, y, acc)
    oh = (idx[:, None] ==
          jax.lax.broadcasted_iota(jnp.int32, (_TB, _NUM_EXPERTS), 1))
    bias = jnp.dot(oh.astype(jnp.float32), be_ref[...],
                   preferred_element_type=jnp.float32)
    out_ref[...] = gate_val[:, None] * (acc + bias)


@jax.jit
def kernel(x, Wg, We, be):
    B, S, H = x.shape
    T = B * S
    xt = x.reshape(T, H)
    grid = (T // _TB,)
    out = pl.pallas_call(
        _moe_block_kernel,
        grid=grid,
        in_specs=[
            pl.BlockSpec((_TB, H), lambda i: (i, 0)),
            pl.BlockSpec((H, _NUM_EXPERTS), lambda i: (0, 0)),
            pl.BlockSpec((_NUM_EXPERTS, H, H), lambda i: (0, 0, 0)),
            pl.BlockSpec((_NUM_EXPERTS, H), lambda i: (0, 0)),
        ],
        out_specs=pl.BlockSpec((_TB, H), lambda i: (i, 0)),
        out_shape=jax.ShapeDtypeStruct((T, H), jnp.float32),
    )(xt, Wg, We, be)
    return out.reshape(B, S, H)
